# baseline (speedup 1.0000x reference)
import jax
import jax.numpy as jnp
from jax.experimental import pallas as pl

_NUM_TO_REPLACE = 64
_B, _N, _C = 32, 8192, 128
_ROWS = _B * _N
_BLOCK_ROWS = _N // 4


def _mask_rows():
    perm = jax.random.permutation(jax.random.key(42), _N)
    i_to_zero = perm[:_NUM_TO_REPLACE]
    hit = jnp.any(jnp.arange(_N)[:, None] == i_to_zero[None, :], axis=1)
    m = jnp.where(hit, 0.0, 1.0).astype(jnp.float32)[:, None]
    return jnp.tile(m, (max(1, _BLOCK_ROWS // _N), 1))


def _body(pts_ref, mask_ref, out_ref):
    out_ref[...] = pts_ref[...] * mask_ref[...]


def kernel(pts):
    flat = pts.reshape(_ROWS, _C)
    mask = _mask_rows()
    out = pl.pallas_call(
        _body,
        grid=(_ROWS // _BLOCK_ROWS,),
        in_specs=[
            pl.BlockSpec((_BLOCK_ROWS, _C), lambda i: (i, 0)),
            pl.BlockSpec(
                (_BLOCK_ROWS, 1),
                lambda i: (i % max(1, _N // _BLOCK_ROWS), 0),
            ),
        ],
        out_specs=pl.BlockSpec((_BLOCK_ROWS, _C), lambda i: (i, 0)),
        out_shape=jax.ShapeDtypeStruct((_ROWS, _C), jnp.float32),
    )(flat, mask)
    return out.reshape(_B, _N, _C)


# speedup vs baseline: 1.6051x; 1.6051x over previous
import jax
import jax.numpy as jnp
from jax.experimental import pallas as pl

_NUM_TO_REPLACE = 64
_B, _N, _C = 32, 8192, 128
_ROWS = _B * _N
_BLOCK_ROWS = _N


def _mask_rows():
    perm = jax.random.permutation(jax.random.key(42), _N)
    i_to_zero = perm[:_NUM_TO_REPLACE]
    hit = jnp.any(jnp.arange(_N)[:, None] == i_to_zero[None, :], axis=1)
    m = jnp.where(hit, 0.0, 1.0).astype(jnp.float32)[:, None]
    return jnp.tile(m, (max(1, _BLOCK_ROWS // _N), 1))


def _body(pts_ref, mask_ref, out_ref):
    out_ref[...] = pts_ref[...] * mask_ref[...]


def kernel(pts):
    flat = pts.reshape(_ROWS, _C)
    mask = _mask_rows()
    out = pl.pallas_call(
        _body,
        grid=(_ROWS // _BLOCK_ROWS,),
        in_specs=[
            pl.BlockSpec((_BLOCK_ROWS, _C), lambda i: (i, 0)),
            pl.BlockSpec(
                (_BLOCK_ROWS, 1),
                lambda i: (i % max(1, _N // _BLOCK_ROWS), 0),
            ),
        ],
        out_specs=pl.BlockSpec((_BLOCK_ROWS, _C), lambda i: (i, 0)),
        out_shape=jax.ShapeDtypeStruct((_ROWS, _C), jnp.float32),
    )(flat, mask)
    return out.reshape(_B, _N, _C)
